# Initial kernel scaffold; baseline (speedup 1.0000x reference)
import jax
import jax.numpy as jnp
from jax.experimental import pallas as pl


def kernel(x, edge, W1, a_src1, a_dst1, b1, W2, a_src2, a_dst2, b2):
    raise NotImplementedError("write your pallas kernel here")


# speedup vs baseline: 24.9847x; 24.9847x over previous
import functools

import jax
import jax.numpy as jnp
from jax import lax
from jax.experimental import pallas as pl
from jax.experimental.pallas import tpu as pltpu
from jax.experimental.pallas import tpu_sc as plsc

N = 10000
E = 320000
IN = 128
H = 8
F1 = 8
OUT = 40
D1 = H * F1
D2 = H * OUT

K = 128
NSUB = 16
CHUNKS = 157
ESUB = CHUNKS * K
EP = ESUB * NSUB
ACC_ROWS = 10112
R_INIT = ACC_ROWS // NSUB
R_OUT = ACC_ROWS // NSUB


def _sc_edge_body(f_half, fd, stage, src_r, dst_r, h_t, as_t, ad_t, numi_r, deni_r,
                  num_o, den_o,
                  acc, dacc, idx_s, idx_hs, idx_d, as_rows, ad_rows, ex_buf,
                  h_rows, sem):
    c = lax.axis_index("c")
    s = lax.axis_index("s")
    n_slots = f_half // 16
    r0 = s * R_INIT
    pltpu.sync_copy(numi_r.at[pl.ds(c * ACC_ROWS + r0, R_INIT)],
                    acc.at[pl.ds(r0, R_INIT)])
    pltpu.sync_copy(deni_r.at[pl.ds(r0, R_INIT)], dacc.at[pl.ds(r0, R_INIT)])
    plsc.subcore_barrier()

    lanes = lax.iota(jnp.int32, 16)
    headmaps = [(c * f_half + 16 * j + lanes) // fd for j in range(n_slots)]
    e0 = s * ESUB

    def chunk(i, carry):
        base = e0 + i * K
        pltpu.sync_copy(src_r.at[pl.ds(base, K)], idx_s)
        pltpu.sync_copy(dst_r.at[pl.ds(base, K)], idx_d)
        if stage >= 3:
            for q in range(K // 16):
                idx_hs[pl.ds(16 * q, 16)] = idx_s[pl.ds(16 * q, 16)] + c * N
        if stage >= 4:
            pltpu.async_copy(as_t.at[idx_s], as_rows, sem).wait()
            pltpu.async_copy(ad_t.at[idx_d], ad_rows, sem).wait()
            pltpu.async_copy(h_t.at[idx_hs], h_rows, sem).wait()
        ex_mode = 4 if stage not in (50, 51, 52, 53) else stage - 50
        if stage >= 5:
            for q in range(K * H // 16):
                row = (16 * q + lanes) // H
                col = (16 * q + lanes) % H
                if ex_mode == 0:
                    a = jnp.full((16,), 1.0, jnp.float32)
                elif ex_mode == 1:
                    a = plsc.load_gather(as_rows, [row, col])
                else:
                    a = (plsc.load_gather(as_rows, [row, col]) +
                         plsc.load_gather(ad_rows, [row, col]))
                if ex_mode >= 3:
                    a = jnp.where(a > 0, a, 0.2 * a)
                if ex_mode >= 4:
                    a = jnp.exp(a)
                plsc.store_scatter(ex_buf, [row, col], a)

        def edge(k, carry2):
            kv = jnp.full((16,), 0, jnp.int32) + k
            for j in range(n_slots):
                colv = 16 * j + lanes
                w = plsc.load_gather(ex_buf, [kv, headmaps[j]])
                hv = plsc.load_gather(h_rows, [kv, colv])
                plsc.store_scatter(h_rows, [kv, colv], hv * w)
            return carry2

        if stage >= 6:
            lax.fori_loop(0, K, edge, 0)
        if stage >= 7:
            pltpu.sync_copy(h_rows, acc.at[idx_d], add=True)
            pltpu.sync_copy(ex_buf, dacc.at[idx_d], add=True)
        return carry

    if stage >= 2:
        lax.fori_loop(0, CHUNKS, chunk, 0)
    plsc.subcore_barrier()
    w0 = s * R_OUT
    pltpu.sync_copy(acc.at[pl.ds(w0, R_OUT)],
                    num_o.at[pl.ds(c * ACC_ROWS + w0, R_OUT)])
    pltpu.sync_copy(dacc.at[pl.ds(w0, R_OUT)],
                    den_o.at[pl.ds(c * ACC_ROWS + w0, R_OUT)])


@functools.lru_cache(maxsize=None)
def _make_sc_edge(f_half, fd):
    return _make_sc_edge_staged(f_half, fd, 99)


@functools.lru_cache(maxsize=None)
def _make_sc_edge_staged(f_half, fd, stage):
    mesh = plsc.VectorSubcoreMesh(core_axis_name="c", subcore_axis_name="s",
                                  num_cores=2, num_subcores=NSUB)
    return pl.kernel(
        functools.partial(_sc_edge_body, f_half, fd, stage),
        compiler_params=pltpu.CompilerParams(use_tc_tiling_on_sc=False, needs_layout_passes=False),
        out_type=[jax.ShapeDtypeStruct((2 * ACC_ROWS, f_half), jnp.float32),
                  jax.ShapeDtypeStruct((2 * ACC_ROWS, H), jnp.float32)],
        mesh=mesh,
        scratch_types=[
            pltpu.VMEM_SHARED((ACC_ROWS, f_half), jnp.float32),
            pltpu.VMEM_SHARED((ACC_ROWS, H), jnp.float32),
            pltpu.VMEM((K,), jnp.int32),
            pltpu.VMEM((K,), jnp.int32),
            pltpu.VMEM((K,), jnp.int32),
            pltpu.VMEM((K, H), jnp.float32),
            pltpu.VMEM((K, H), jnp.float32),
            pltpu.VMEM((K, H), jnp.float32),
            pltpu.VMEM((K, f_half), jnp.float32),
            pltpu.SemaphoreType.DMA,
        ],
    )


def _tc1_body(x_r, w1_r, asm_r, adm_r, h_o, as_o, ad_o, exs_o, numi_o):
    x = x_r[...]
    h = lax.dot_general(x, w1_r[...], (((1,), (1,)), ((), ())),
                        preferred_element_type=jnp.float32)
    a_s = jnp.dot(h, asm_r[...], preferred_element_type=jnp.float32)
    a_d = jnp.dot(h, adm_r[...], preferred_element_type=jnp.float32)
    al = a_s + a_d
    al = jnp.where(al > 0, al, 0.2 * al)
    exs = jnp.exp(al)
    h_o[...] = h
    as_o[...] = a_s
    ad_o[...] = a_d
    exs_o[...] = exs
    numi_o[...] = jnp.concatenate(
        [h[:, F1 * t:F1 * (t + 1)] * exs[:, t:t + 1] for t in range(H)],
        axis=1)


def _tc2_body(num_r, den_r, b1_r, w2_r, asm_r, adm_r,
              h_o, as_o, ad_o, exs_o, numi_o):
    num = num_r[...]
    r = 1.0 / (den_r[...] + 1e-16)
    h1 = jnp.concatenate(
        [num[:, F1 * t:F1 * (t + 1)] * r[:, t:t + 1] for t in range(H)],
        axis=1) + b1_r[...]
    h1 = jnp.where(h1 > 0, h1, jnp.exp(jnp.minimum(h1, 0.0)) - 1.0)
    h = lax.dot_general(h1, w2_r[...], (((1,), (1,)), ((), ())),
                        preferred_element_type=jnp.float32)
    a_s = jnp.dot(h, asm_r[...], preferred_element_type=jnp.float32)
    a_d = jnp.dot(h, adm_r[...], preferred_element_type=jnp.float32)
    al = a_s + a_d
    al = jnp.where(al > 0, al, 0.2 * al)
    exs = jnp.exp(al)
    h_o[...] = h
    as_o[...] = a_s
    ad_o[...] = a_d
    exs_o[...] = exs
    numi_o[...] = jnp.concatenate(
        [h[:, OUT * t:OUT * (t + 1)] * exs[:, t:t + 1] for t in range(H)],
        axis=1)


def _tc3_body(num_r, den_r, b2_r, out_o):
    num = num_r[...]
    r = 0.125 / (den_r[...] + 1e-16)
    acc = num[:, 0:OUT] * r[:, 0:1]
    for t in range(1, H):
        acc = acc + num[:, OUT * t:OUT * (t + 1)] * r[:, t:t + 1]
    out_o[...] = acc + b2_r[...]


_BN = 2000
_GRID = N // _BN


def _row_spec(width):
    return pl.BlockSpec((_BN, width), lambda i: (i, 0))


def _full_spec(shape):
    return pl.BlockSpec(shape, lambda i: tuple(0 for _ in shape))


_tc1 = pl.pallas_call(
    _tc1_body,
    grid=(_GRID,),
    in_specs=[_row_spec(IN), _full_spec((D1, IN)), _full_spec((D1, H)),
              _full_spec((D1, H))],
    out_specs=[_row_spec(D1), _row_spec(H), _row_spec(H), _row_spec(H),
               _row_spec(D1)],
    out_shape=[jax.ShapeDtypeStruct((N, D1), jnp.float32),
               jax.ShapeDtypeStruct((N, H), jnp.float32),
               jax.ShapeDtypeStruct((N, H), jnp.float32),
               jax.ShapeDtypeStruct((N, H), jnp.float32),
               jax.ShapeDtypeStruct((N, D1), jnp.float32)],
)

_tc2 = pl.pallas_call(
    _tc2_body,
    grid=(_GRID,),
    in_specs=[_row_spec(D1), _row_spec(H), _full_spec((1, D1)),
              _full_spec((D2, D1)), _full_spec((D2, H)), _full_spec((D2, H))],
    out_specs=[_row_spec(D2), _row_spec(H), _row_spec(H), _row_spec(H),
               _row_spec(D2)],
    out_shape=[jax.ShapeDtypeStruct((N, D2), jnp.float32),
               jax.ShapeDtypeStruct((N, H), jnp.float32),
               jax.ShapeDtypeStruct((N, H), jnp.float32),
               jax.ShapeDtypeStruct((N, H), jnp.float32),
               jax.ShapeDtypeStruct((N, D2), jnp.float32)],
)

_tc3 = pl.pallas_call(
    _tc3_body,
    grid=(_GRID,),
    in_specs=[_row_spec(D2), _row_spec(H), _full_spec((1, OUT))],
    out_specs=_row_spec(OUT),
    out_shape=jax.ShapeDtypeStruct((N, OUT), jnp.float32),
)


def _expand_att(a):
    hh, ff = a.shape
    eye = jnp.eye(hh, dtype=jnp.float32)
    return (a[:, :, None] * eye[:, None, :]).reshape(hh * ff, hh)


def _split_pad(arr, f_half):
    pad = jnp.zeros((ACC_ROWS - N, f_half), jnp.float32)
    return jnp.concatenate(
        [arr[:, :f_half], pad, arr[:, f_half:], pad], axis=0)


def kernel(x, edge, W1, a_src1, a_dst1, b1, W2, a_src2, a_dst2, b2):
    As1 = _expand_att(a_src1)
    Ad1 = _expand_att(a_dst1)
    As2 = _expand_att(a_src2)
    Ad2 = _expand_att(a_dst2)
    src = jnp.concatenate(
        [edge[0].astype(jnp.int32), jnp.zeros((EP - E,), jnp.int32)])
    dst = jnp.concatenate(
        [edge[1].astype(jnp.int32), jnp.full((EP - E,), N, jnp.int32)])

    h1, as1, ad1, exs1, numi1 = _tc1(x, W1, As1, Ad1)
    h1b = jnp.concatenate([h1[:, :D1 // 2], h1[:, D1 // 2:]], axis=0)
    ad1p = jnp.concatenate([ad1, jnp.zeros((1, H), jnp.float32)], axis=0)
    deni1 = jnp.concatenate(
        [exs1, jnp.zeros((ACC_ROWS - N, H), jnp.float32)], axis=0)
    num1, den1 = _make_sc_edge(D1 // 2, F1)(src, dst, h1b, as1, ad1p,
                                            _split_pad(numi1, D1 // 2), deni1)
    num1c = jnp.concatenate(
        [num1[:N], num1[ACC_ROWS:ACC_ROWS + N]], axis=1)

    h2, as2, ad2, exs2, numi2 = _tc2(num1c, den1[:N], b1.reshape(1, D1),
                                     W2, As2, Ad2)
    h2b = jnp.concatenate([h2[:, :D2 // 2], h2[:, D2 // 2:]], axis=0)
    ad2p = jnp.concatenate([ad2, jnp.zeros((1, H), jnp.float32)], axis=0)
    deni2 = jnp.concatenate(
        [exs2, jnp.zeros((ACC_ROWS - N, H), jnp.float32)], axis=0)
    num2, den2 = _make_sc_edge(D2 // 2, OUT)(src, dst, h2b, as2, ad2p,
                                             _split_pad(numi2, D2 // 2), deni2)
    num2c = jnp.concatenate(
        [num2[:N], num2[ACC_ROWS:ACC_ROWS + N]], axis=1)

    return _tc3(num2c, den2[:N], b2.reshape(1, OUT))
